# Initial kernel scaffold; baseline (speedup 1.0000x reference)
import jax
import jax.numpy as jnp
from jax.experimental import pallas as pl


def kernel(x, edge_index, edge_weight, batch, W1, b1, W2, b2, W3, b3, g1, be1, g2, be2, g3, be3, fW1, fb1, fW2, fb2):
    raise NotImplementedError("write your pallas kernel here")


# speedup vs baseline: 1.0012x; 1.0012x over previous
import functools

import jax
import jax.numpy as jnp
from jax import lax
from jax.experimental import pallas as pl
from jax.experimental.pallas import tpu as pltpu
from jax.experimental.pallas import tpu_sc as plsc

N = 10000
E = 320000
F_IN = 128
H = 256
C = 16
G = 64

N_PAD = 10240
E2 = E + N
K = 128
EPT = 168
E_PAD = 16 * K * EPT
ER = E_PAD // K
HQ = H // 4
RB = 1024
NB = N_PAD // RB

_mesh = plsc.VectorSubcoreMesh(core_axis_name="c", subcore_axis_name="s")

f32 = jnp.float32
i32 = jnp.int32


def _fill_zero(ref, nrows):
    z = jnp.zeros((16,), f32)

    def body(i, _):
        ref[i // 8, pl.ds((i % 8) * 16, 16)] = z
        return 0

    lax.fori_loop(0, nrows * 8, body, 0)


def _deg_body(d_hbm, w_hbm, dinv_hbm, d_v, w_v, wk_v, deg_sh):
    cid = lax.axis_index("c")
    sid = lax.axis_index("s")

    z = jnp.zeros((16,), f32)

    def zb(i, _):
        wk_v[pl.ds(i * 16, 16)] = z
        return 0

    lax.fori_loop(0, 40, zb, 0)
    pltpu.sync_copy(wk_v, deg_sh.at[pl.ds(sid * 640, 640)])
    plsc.subcore_barrier()

    pltpu.sync_copy(d_hbm.at[pl.ds(sid * EPT, EPT)], d_v)
    pltpu.sync_copy(w_hbm.at[pl.ds(sid * EPT, EPT)], w_v)

    def sb(j, _):
        pltpu.sync_copy(w_v.at[j], deg_sh.at[d_v.at[j]], add=True)
        return 0

    lax.fori_loop(0, EPT, sb, 0)
    plsc.subcore_barrier()

    off = cid * 5120 + sid * 320
    pltpu.sync_copy(deg_sh.at[pl.ds(off, 320)], wk_v.at[pl.ds(0, 320)])

    def rb(i, _):
        x = wk_v[pl.ds(i * 16, 16)]
        xi = lax.bitcast_convert_type(x, i32)
        yi = jnp.int32(0x5F3759DF) - (xi >> 1)
        y = lax.bitcast_convert_type(yi, f32)
        y = y * (1.5 - 0.5 * x * y * y)
        y = y * (1.5 - 0.5 * x * y * y)
        y = y * (1.5 - 0.5 * x * y * y)
        wk_v[pl.ds(i * 16, 16)] = jnp.where(x > 0.0, y, 0.0)
        return 0

    lax.fori_loop(0, 20, rb, 0)
    pltpu.sync_copy(wk_v.at[pl.ds(0, 320)], dinv_hbm.at[pl.ds(off, 320)])


_deg_kernel = functools.partial(
    pl.kernel,
    out_type=jax.ShapeDtypeStruct((N_PAD,), f32),
    mesh=_mesh,
    scratch_types=[
        pltpu.VMEM((EPT, K), i32),
        pltpu.VMEM((EPT, K), f32),
        pltpu.VMEM((640,), f32),
        pltpu.VMEM_SHARED((N_PAD,), f32),
    ],
)(_deg_body)


ACC_R = 2624
DR = 2560


def _conv_body(r, tbl, s_hbm, d_hbm, w_hbm, out,
               s_v, d_v, w_v, dl_v, rows0, rows1, sem0, sem1, acc_sh):
    cid = lax.axis_index("c")
    sid = lax.axis_index("s")
    base = (2 * r + cid) * DR

    _fill_zero(rows0, 128)

    @pl.when(sid < 8)
    def _():
        pltpu.sync_copy(rows0, acc_sh.at[pl.ds(sid * 328, 128)])
        pltpu.sync_copy(rows0, acc_sh.at[pl.ds(sid * 328 + 128, 128)])
        pltpu.sync_copy(rows0.at[pl.ds(0, 72)],
                        acc_sh.at[pl.ds(sid * 328 + 256, 72)])

    plsc.subcore_barrier()

    pltpu.sync_copy(s_hbm.at[pl.ds(sid * EPT, EPT)], s_v)
    pltpu.sync_copy(d_hbm.at[pl.ds(sid * EPT, EPT)], d_v)
    pltpu.sync_copy(w_hbm.at[pl.ds(sid * EPT, EPT)], w_v)

    def issue(j, buf, sem):
        pltpu.async_copy(tbl.at[s_v.at[j]], buf, sem)

    def process(j, buf, sem):
        pltpu.make_async_copy(tbl.at[s_v.at[j]], buf, sem).wait()

        def scale(gg, _):
            wvec = w_v[j, pl.ds(gg * 16, 16)]
            dvec = d_v[j, pl.ds(gg * 16, 16)] - base
            ok = jnp.logical_and(dvec >= 0, dvec < DR)
            dl_v[pl.ds(gg * 16, 16)] = jnp.where(ok, dvec, DR)
            for l in range(16):
                e = gg * 16 + l
                wb = jnp.full((16,), wvec[l], f32)
                for f in range(8):
                    buf[e, pl.ds(f * 16, 16)] = buf[e, pl.ds(f * 16, 16)] * wb
            return 0

        lax.fori_loop(0, K // 16, scale, 0)
        pltpu.sync_copy(buf, acc_sh.at[dl_v], add=True)

    issue(0, rows0, sem0)

    def loop(i, _):
        issue(2 * i + 1, rows1, sem1)
        process(2 * i, rows0, sem0)

        @pl.when(i < EPT // 2 - 1)
        def _():
            issue(2 * i + 2, rows0, sem0)

        process(2 * i + 1, rows1, sem1)
        return 0

    lax.fori_loop(0, EPT // 2, loop, 0)
    plsc.subcore_barrier()

    pltpu.sync_copy(acc_sh.at[pl.ds(sid * 160, 160)],
                    out.at[pl.ds(cid * DR + sid * 160, 160)])


def _make_conv(r):
    return functools.partial(
        pl.kernel,
        out_type=jax.ShapeDtypeStruct((2 * DR, 128), f32),
        mesh=_mesh,
        scratch_types=[
        pltpu.VMEM((EPT, K), i32),
        pltpu.VMEM((EPT, K), i32),
        pltpu.VMEM((EPT, K), f32),
        pltpu.VMEM((K,), i32),
        pltpu.VMEM((K, 128), f32),
        pltpu.VMEM((K, 128), f32),
        pltpu.SemaphoreType.DMA,
        pltpu.SemaphoreType.DMA,
        pltpu.VMEM_SHARED((ACC_R, 128), f32),
    ],
    )(functools.partial(_conv_body, r))


_conv_r0 = _make_conv(0)
_conv_r1 = _make_conv(1)


def _halves(r, lo, hi):
    lo[...] = r[:, :128]
    hi[...] = r[:, 128:]


def _m1_body(x_ref, dinv_ref, w_ref, olo, ohi):
    r = jnp.dot(x_ref[...], w_ref[...], preferred_element_type=f32)
    r = r * dinv_ref[...]
    _halves(r, olo, ohi)


def _m1(x_p, dinv2, W1):
    return pl.pallas_call(
        _m1_body,
        grid=(NB,),
        in_specs=[
            pl.BlockSpec((RB, F_IN), lambda j: (j, 0)),
            pl.BlockSpec((RB, 1), lambda j: (j, 0)),
            pl.BlockSpec((F_IN, H), lambda j: (0, 0)),
        ],
        out_specs=[pl.BlockSpec((RB, 128), lambda j: (j, 0))] * 2,
        out_shape=[jax.ShapeDtypeStruct((N_PAD, 128), f32)] * 2,
    )(x_p, dinv2, W1)


def _bn_block(aggs, dinv, b, g, be, stats, p, j):
    conv = jnp.concatenate([a[...] for a in aggs], axis=1) * dinv[...] \
        + b[0:1, :]

    @pl.when(jnp.logical_and(p == 0, j == 0))
    def _():
        stats[...] = jnp.zeros_like(stats)

    @pl.when(p == 0)
    def _():
        valid = N - j * RB
        rows = lax.broadcasted_iota(i32, (RB, H), 0)
        cm = jnp.where(rows < valid, conv, 0.0)
        stats[0:1, :] += jnp.sum(cm, axis=0, keepdims=True)
        stats[1:2, :] += jnp.sum(cm * cm, axis=0, keepdims=True)

    mean = stats[0:1, :] * (1.0 / N)
    var = stats[1:2, :] * (1.0 / N) - mean * mean
    h = jnp.maximum((conv - mean) * lax.rsqrt(var + 1e-5) * g[0:1, :]
                    + be[0:1, :], 0.0)
    return h


def _bmid_body(alo, ahi, dinv, b, g, be, wn, olo, ohi, stats):
    p = pl.program_id(0)
    j = pl.program_id(1)
    h = _bn_block((alo, ahi), dinv, b, g, be, stats, p, j)

    @pl.when(p == 1)
    def _():
        r = jnp.dot(h, wn[...], preferred_element_type=f32) * dinv[...]
        _halves(r, olo, ohi)


def _bmid(aggs, dinv2, br, gr, ber, Wn):
    return pl.pallas_call(
        _bmid_body,
        grid=(2, NB),
        in_specs=[pl.BlockSpec((RB, 128), lambda p, j: (j, 0))] * 2 + [
            pl.BlockSpec((RB, 1), lambda p, j: (j, 0)),
            pl.BlockSpec((8, H), lambda p, j: (0, 0)),
            pl.BlockSpec((8, H), lambda p, j: (0, 0)),
            pl.BlockSpec((8, H), lambda p, j: (0, 0)),
            pl.BlockSpec((H, H), lambda p, j: (0, 0)),
        ],
        out_specs=[pl.BlockSpec((RB, 128), lambda p, j: (j, 0))] * 2,
        out_shape=[jax.ShapeDtypeStruct((N_PAD, 128), f32)] * 2,
        scratch_shapes=[pltpu.VMEM((8, H), f32)],
    )(*aggs, dinv2, br, gr, ber, Wn)


def _b3_body(alo, ahi, dinv, b, g, be, batchf, fw1, fb1, fw2, fb2,
             out, stats, xsum, cnt):
    p = pl.program_id(0)
    j = pl.program_id(1)
    h = _bn_block((alo, ahi), dinv, b, g, be, stats, p, j)

    @pl.when(jnp.logical_and(p == 0, j == 0))
    def _():
        xsum[...] = jnp.zeros_like(xsum)
        cnt[...] = jnp.zeros_like(cnt)

    @pl.when(p == 1)
    def _():
        lanes = lax.broadcasted_iota(i32, (RB, 2 * G), 1).astype(f32)
        onehot = (batchf[...] == lanes).astype(f32)
        xsum[...] += lax.dot_general(onehot, h, (((0,), (0,)), ((), ())),
                                     preferred_element_type=f32)
        cnt[:, 0:1] += lax.dot_general(onehot, jnp.ones((RB, 1), f32),
                                       (((0,), (0,)), ((), ())),
                                       preferred_element_type=f32)

    @pl.when(jnp.logical_and(p == 1, j == NB - 1))
    def _():
        xs = xsum[...]
        mean = xs * (1.0 / jnp.maximum(cnt[:, 0:1], 1.0))
        z = jnp.concatenate([mean, xs], axis=1)
        t = jnp.maximum(jnp.dot(z, fw1[...], preferred_element_type=f32)
                        + fb1[0:1, :], 0.0)
        out[...] = (jnp.dot(t, fw2[...], preferred_element_type=f32)
                    + fb2[0:1, :])


def _b3(aggs, dinv2, br, gr, ber, batchf, fW1, fb1r, fW2p, fb2r):
    return pl.pallas_call(
        _b3_body,
        grid=(2, NB),
        in_specs=[pl.BlockSpec((RB, 128), lambda p, j: (j, 0))] * 2 + [
            pl.BlockSpec((RB, 1), lambda p, j: (j, 0)),
            pl.BlockSpec((8, H), lambda p, j: (0, 0)),
            pl.BlockSpec((8, H), lambda p, j: (0, 0)),
            pl.BlockSpec((8, H), lambda p, j: (0, 0)),
            pl.BlockSpec((RB, 1), lambda p, j: (j, 0)),
            pl.BlockSpec((2 * H, H), lambda p, j: (0, 0)),
            pl.BlockSpec((8, H), lambda p, j: (0, 0)),
            pl.BlockSpec((H, 2 * G), lambda p, j: (0, 0)),
            pl.BlockSpec((8, 2 * G), lambda p, j: (0, 0)),
        ],
        out_specs=pl.BlockSpec((2 * G, 2 * G), lambda p, j: (0, 0)),
        out_shape=jax.ShapeDtypeStruct((2 * G, 2 * G), f32),
        scratch_shapes=[pltpu.VMEM((8, H), f32),
                        pltpu.VMEM((2 * G, H), f32),
                        pltpu.VMEM((2 * G, 8), f32)],
    )(*aggs, dinv2, br, gr, ber, batchf, fW1, fb1r, fW2p, fb2r)


def kernel(x, edge_index, edge_weight, batch, W1, b1, W2, b2, W3, b3,
           g1, be1, g2, be2, g3, be3, fW1, fb1, fW2, fb2):
    loop = jnp.arange(N, dtype=jnp.int32)
    s2 = jnp.concatenate([edge_index[0], loop])
    d2 = jnp.concatenate([edge_index[1], loop])
    w2 = jnp.concatenate([edge_weight, jnp.ones((N,), f32)])
    pad = E_PAD - E2
    s2 = jnp.pad(s2, (0, pad)).reshape(ER, K)
    d2 = jnp.pad(d2, (0, pad)).reshape(ER, K)
    w2 = jnp.pad(w2, (0, pad)).reshape(ER, K)

    x_p = jnp.pad(x, ((0, N_PAD - N), (0, 0)))
    batchf = jnp.pad(batch.astype(f32), (0, N_PAD - N),
                     constant_values=1e9).reshape(N_PAD, 1)

    def row8(v):
        return jnp.broadcast_to(v[None, :], (8, v.shape[0]))

    b1r, g1r, be1r = row8(b1), row8(g1), row8(be1)
    b2r, g2r, be2r = row8(b2), row8(g2), row8(be2)
    b3r, g3r, be3r = row8(b3), row8(g3), row8(be3)
    fb1r = row8(fb1)
    fW2p = jnp.pad(fW2, ((0, 0), (0, 2 * G - C)))
    fb2r = row8(jnp.pad(fb2, (0, 2 * G - C)))

    dinv = _deg_kernel(d2, w2)
    dinv2 = dinv.reshape(N_PAD, 1)

    def agg(hq):
        def half(t):
            a = _conv_r0(t, s2, d2, w2)
            b = _conv_r1(t, s2, d2, w2)
            return jnp.concatenate([a, b], axis=0)
        return (half(hq[0]), half(hq[1]))

    hq = _m1(x_p, dinv2, W1)
    hq = _bmid(agg(hq), dinv2, b1r, g1r, be1r, W2)
    hq = _bmid(agg(hq), dinv2, b2r, g2r, be2r, W3)
    out = _b3(agg(hq), dinv2, b3r, g3r, be3r, batchf, fW1, fb1r, fW2p, fb2r)
    return out[:G, :C]
